# baseline (speedup 1.0000x reference)
import functools

import jax
import jax.numpy as jnp
from jax.experimental import pallas as pl
from jax.experimental.pallas import tpu as pltpu
from jax.experimental.pallas import tpu_sc as plsc

_VOCAB = 1000
_SC_CORES = 2
_SC_SUBCORES = 16
_SC_LANES = 16
_SC_WIN = 256


def _prep_kernel(table_ref, tabt_ref, lse_ref):
    tab = table_ref[...]
    m = jnp.max(tab, axis=1, keepdims=True)
    lse = m + jnp.log(jnp.sum(jnp.exp(tab - m), axis=1, keepdims=True))
    tabt_ref[...] = tab.astype(jnp.bfloat16).T
    lse_ref[...] = lse.T


def _tc_kernel(idx_ref, tabt_ref, out_ref):
    nb = out_ref.shape[2]
    idx_row = idx_ref[0, 0, :]
    viota = jax.lax.broadcasted_iota(jnp.int32, (_VOCAB, nb), 0)
    onehot_t = (viota == idx_row[None, :]).astype(jnp.bfloat16)
    out_ref[0] = jnp.dot(tabt_ref[...], onehot_t,
                         preferred_element_type=jnp.float32)


def _sc_loss_partials(table_flat, lse_flat, pick_idx, lse_idx):
    nidx = pick_idx.shape[1]
    mesh = plsc.VectorSubcoreMesh(core_axis_name="core",
                                  subcore_axis_name="subcore")

    @pl.kernel(
        out_type=jax.ShapeDtypeStruct(
            (2, _SC_CORES, _SC_SUBCORES, _SC_LANES), jnp.float32),
        mesh=mesh,
        scratch_types=[pltpu.VMEM((_SC_WIN,), jnp.float32),
                       pltpu.VMEM((_SC_LANES,), jnp.float32)],
    )
    def kern(tab_hbm, lse_hbm, pidx_hbm, lidx_hbm, o_hbm, gath_vmem, acc_vmem):
        core = jax.lax.axis_index("core")
        sub = jax.lax.axis_index("subcore")

        def accumulate(i_vmem, src_hbm):
            pltpu.sync_copy(src_hbm.at[i_vmem.at[0]], gath_vmem)

            @pl.loop(0, _SC_WIN, step=_SC_LANES)
            def _(c):
                acc_vmem[...] += gath_vmem[pl.ds(c, _SC_LANES)]

        acc_vmem[...] = jnp.zeros((_SC_LANES,), jnp.float32)
        pltpu.emit_pipeline(
            lambda i_vmem: accumulate(i_vmem, tab_hbm),
            grid=(nidx // _SC_WIN,),
            in_specs=[pl.BlockSpec((1, _SC_WIN), index_map=lambda i: (0, i))],
            out_specs=[],
            core_axis_name=("core", "subcore"),
            dimension_semantics=(pltpu.PARALLEL,),
        )(pidx_hbm)
        pltpu.sync_copy(acc_vmem, o_hbm.at[0, core, sub])

        acc_vmem[...] = jnp.zeros((_SC_LANES,), jnp.float32)
        pltpu.emit_pipeline(
            lambda i_vmem: accumulate(i_vmem, lse_hbm),
            grid=(nidx // _SC_WIN,),
            in_specs=[pl.BlockSpec((1, _SC_WIN), index_map=lambda i: (0, i))],
            out_specs=[],
            core_axis_name=("core", "subcore"),
            dimension_semantics=(pltpu.PARALLEL,),
        )(lidx_hbm)
        pltpu.sync_copy(acc_vmem, o_hbm.at[1, core, sub])

    return kern(table_flat, lse_flat, pick_idx, lse_idx)


@functools.partial(jax.jit, static_argnames=())
def kernel(table, index, targets):
    b, t = index.shape
    n = b * t
    idx32 = index.astype(jnp.int32)
    idx = idx32.T.reshape(t, 1, b)

    tabt, lse_row = pl.pallas_call(
        _prep_kernel,
        out_shape=[
            jax.ShapeDtypeStruct((_VOCAB, _VOCAB), jnp.bfloat16),
            jax.ShapeDtypeStruct((1, _VOCAB), jnp.float32),
        ],
    )(table)

    pick_idx = (idx32 * _VOCAB + targets.astype(jnp.int32)).reshape(1, n)
    lse_idx = idx32.reshape(1, n)
    partials = _sc_loss_partials(table.reshape(_VOCAB * _VOCAB),
                                 lse_row.reshape(_VOCAB), pick_idx, lse_idx)

    logits_t = pl.pallas_call(
        _tc_kernel,
        grid=(t,),
        in_specs=[
            pl.BlockSpec((1, 1, b), lambda i: (i, 0, 0)),
            pl.BlockSpec((_VOCAB, _VOCAB), lambda i: (0, 0)),
        ],
        out_specs=pl.BlockSpec((1, _VOCAB, b), lambda i: (i, 0, 0)),
        out_shape=jax.ShapeDtypeStruct((t, _VOCAB, b), jnp.float32),
    )(idx, tabt)

    logits = jnp.transpose(logits_t, (2, 0, 1))
    loss = (jnp.sum(partials[1]) - jnp.sum(partials[0])) / n
    return (logits, loss)
